# baseline (speedup 1.0000x reference)
import functools

import jax
import jax.numpy as jnp
from jax import lax
from jax.experimental import pallas as pl
from jax.experimental.pallas import tpu as pltpu
from jax.experimental.pallas import tpu_sc as plsc

_NC = 2
_NS = 16
_NW = _NC * _NS
_CHUNK = 128


@functools.lru_cache(maxsize=None)
def _make_gather(V, D, B):
    b_per_w = B // _NW
    n_chunks = b_per_w // _CHUNK
    mesh = plsc.VectorSubcoreMesh(core_axis_name="c", subcore_axis_name="s")

    @functools.partial(
        pl.kernel,
        out_type=jax.ShapeDtypeStruct((B, D), jnp.float32),
        mesh=mesh,
        scratch_types=[
            pltpu.VMEM((n_chunks, _CHUNK), jnp.int32),
            pltpu.VMEM((b_per_w, D), jnp.float32),
            pltpu.SemaphoreType.DMA,
        ],
        compiler_params=pltpu.CompilerParams(use_tc_tiling_on_sc=False),
    )
    def gather(table_hbm, idx_hbm, out_hbm, idx_v, rows_v, sem):
        wid = lax.axis_index("s") * _NC + lax.axis_index("c")
        pltpu.sync_copy(idx_hbm.at[pl.ds(wid * n_chunks, n_chunks)], idx_v)
        copies = [
            pltpu.async_copy(
                table_hbm.at[idx_v.at[j]],
                rows_v.at[pl.ds(j * _CHUNK, _CHUNK)],
                sem,
            )
            for j in range(n_chunks)
        ]
        for c in copies:
            c.wait()
        pltpu.sync_copy(rows_v, out_hbm.at[pl.ds(wid * b_per_w, b_per_w)])

    return gather


def _mlp_body(emb, t, c, w1e, w1t, w1c, b1, w2, b2, out):
    h = jnp.dot(emb[...], w1e[...], preferred_element_type=jnp.float32)
    h = h + jnp.dot(t[...], w1t[...], preferred_element_type=jnp.float32)
    h = h + jnp.dot(c[...], w1c[...], preferred_element_type=jnp.float32)
    h = jnp.maximum(h + b1[...], 0.0)
    out[...] = jnp.sum(h * w2[...], axis=1, keepdims=True) + b2[...]


@functools.lru_cache(maxsize=None)
def _make_mlp(B, D, T, C, H, blk):
    grid = B // blk
    full = lambda shape: pl.BlockSpec(shape, lambda i: (0, 0))
    rows = lambda w: pl.BlockSpec((blk, w), lambda i: (i, 0))
    return pl.pallas_call(
        _mlp_body,
        grid=(grid,),
        in_specs=[
            rows(D), rows(T), rows(C),
            full((D, H)), full((T, H)), full((C, H)),
            full((1, H)), full((1, H)), full((1, 1)),
        ],
        out_specs=rows(1),
        out_shape=jax.ShapeDtypeStruct((B, 1), jnp.float32),
    )


def kernel(seg_idx, temporal, context_flags, table, W1, b1, W2, b2):
    B = seg_idx.shape[0]
    V, D = table.shape
    T = temporal.shape[1]
    C = context_flags.shape[1]
    H = W1.shape[1]

    idx = seg_idx.astype(jnp.int32).reshape(B // _CHUNK, _CHUNK)
    emb = _make_gather(V, D, B)(table, idx)

    out = _make_mlp(B, D, T, C, H, 2048)(
        emb, temporal, context_flags,
        W1[:D], W1[D:D + T], W1[D + T:],
        b1.reshape(1, H), W2.reshape(1, H), b2.reshape(1, 1),
    )
    return out
